# baseline (speedup 1.0000x reference)
import functools

import jax
import jax.numpy as jnp
from jax import lax
from jax.experimental import pallas as pl
from jax.experimental.pallas import tpu as pltpu
from jax.experimental.pallas import tpu_sc as plsc

N_NODES = 10000
N_EDGES = 160000
D_FEAT = 256
D_EDGE = 16
D_GLOBAL = 128

NC = 2
NS = 16
NW = NC * NS
CHUNK = 128
N_CHUNKS = N_EDGES // CHUNK
CHUNKS_FULL = 40
CHUNKS_LAST = N_CHUNKS - (NW - 1) * CHUNKS_FULL
SLAB = CHUNKS_FULL * CHUNK
N_PAD = 10240
ROWS_PER_TILE = N_PAD // NS


def _sc_segment_sum(edges, receivers, zeros):
    mesh = plsc.VectorSubcoreMesh(core_axis_name="c", subcore_axis_name="s")

    @functools.partial(
        pl.kernel,
        out_type=jax.ShapeDtypeStruct((NC, N_PAD, D_EDGE), jnp.float32),
        mesh=mesh,
        scratch_types=[
            pltpu.VMEM((SLAB, D_EDGE), jnp.float32),
            pltpu.VMEM((SLAB,), jnp.int32),
            pltpu.VMEM_SHARED((N_PAD, D_EDGE), jnp.float32),
            pltpu.SemaphoreType.DMA,
            pltpu.SemaphoreType.DMA,
        ],
        compiler_params=pltpu.CompilerParams(use_tc_tiling_on_sc=False),
    )
    def k(edges_hbm, edgesT_hbm, recv_hbm, zeros_hbm, out_hbm, edges_v, idx_v,
          acc_sh, stage_sem, scat_sem):
        c = lax.axis_index("c")
        s = lax.axis_index("s")
        wid = s * NC + c
        is_last = wid == NW - 1
        n_chunks = jnp.where(is_last, CHUNKS_LAST, CHUNKS_FULL)

        zero_cp = pltpu.async_copy(
            zeros_hbm.at[pl.ds(s * ROWS_PER_TILE, ROWS_PER_TILE)],
            acc_sh.at[pl.ds(s * ROWS_PER_TILE, ROWS_PER_TILE)],
            stage_sem,
        )
        pltpu.sync_copy(
            edgesT_hbm.at[:, pl.ds(wid * 16, 16)],
            edges_v.at[pl.ds(0, 16)],
        )

        @pl.when(jnp.logical_not(is_last))
        def _():
            e_cp = pltpu.async_copy(
                edges_hbm.at[pl.ds(wid * SLAB, SLAB)], edges_v, stage_sem
            )
            i_cp = pltpu.async_copy(
                recv_hbm.at[pl.ds(wid * SLAB, SLAB)], idx_v, stage_sem
            )
            e_cp.wait()
            i_cp.wait()

        @pl.when(is_last)
        def _():
            e_cp = pltpu.async_copy(
                edges_hbm.at[pl.ds(wid * SLAB, CHUNKS_LAST * CHUNK)],
                edges_v.at[pl.ds(0, CHUNKS_LAST * CHUNK)],
                stage_sem,
            )
            i_cp = pltpu.async_copy(
                recv_hbm.at[pl.ds(wid * SLAB, CHUNKS_LAST * CHUNK)],
                idx_v.at[pl.ds(0, CHUNKS_LAST * CHUNK)],
                stage_sem,
            )
            e_cp.wait()
            i_cp.wait()

        zero_cp.wait()

        plsc.subcore_barrier()

        @pl.loop(0, n_chunks)
        def _(j):
            pltpu.async_copy(
                edges_v.at[pl.ds(j * CHUNK, CHUNK)],
                acc_sh.at[idx_v.at[pl.ds(j * CHUNK, CHUNK)]],
                scat_sem,
                add=True,
            )

        @pl.loop(0, n_chunks)
        def _(j):
            pltpu.make_async_copy(
                edges_v.at[pl.ds(j * CHUNK, CHUNK)],
                acc_sh.at[idx_v.at[pl.ds(j * CHUNK, CHUNK)]],
                scat_sem,
            ).wait()

        plsc.subcore_barrier()

        pltpu.sync_copy(
            acc_sh.at[pl.ds(s * ROWS_PER_TILE, ROWS_PER_TILE)],
            out_hbm.at[c, pl.ds(s * ROWS_PER_TILE, ROWS_PER_TILE)],
        )

    return k(edges, edges.T, receivers, zeros)


BLK = 1000


def _tc_mlp(partials, nodes, globals_, W, b2):
    def body(p_ref, nodes_ref, g_ref, w_ref, b_ref, o_ref):
        agg = p_ref[0] + p_ref[1]
        w = w_ref[...]
        acc = jnp.dot(agg, w[:D_EDGE], preferred_element_type=jnp.float32)
        acc += jnp.dot(nodes_ref[...], w[D_EDGE:D_EDGE + D_FEAT],
                       preferred_element_type=jnp.float32)
        acc += jnp.dot(g_ref[...], w[D_EDGE + D_FEAT:],
                       preferred_element_type=jnp.float32)
        o_ref[...] = jnp.maximum(acc + b_ref[...], 0.0)

    return pl.pallas_call(
        body,
        grid=(N_NODES // BLK,),
        in_specs=[
            pl.BlockSpec((NC, BLK, D_EDGE), lambda i: (0, i, 0)),
            pl.BlockSpec((BLK, D_FEAT), lambda i: (i, 0)),
            pl.BlockSpec((1, D_GLOBAL), lambda i: (0, 0)),
            pl.BlockSpec((D_EDGE + D_FEAT + D_GLOBAL, D_FEAT),
                         lambda i: (0, 0)),
            pl.BlockSpec((1, D_FEAT), lambda i: (0, 0)),
        ],
        out_specs=pl.BlockSpec((BLK, D_FEAT), lambda i: (i, 0)),
        out_shape=jax.ShapeDtypeStruct((N_NODES, D_FEAT), jnp.float32),
    )(partials, nodes, globals_, W, b2)


def kernel(nodes, edges, receivers, senders, globals_, W, b):
    del senders
    zeros = jnp.zeros((N_PAD, D_EDGE), jnp.float32)
    partials = _sc_segment_sum(edges, receivers, zeros)
    return _tc_mlp(partials, nodes, globals_, W, b.reshape(1, D_FEAT))


# speedup vs baseline: 1.0347x; 1.0347x over previous
import functools

import jax
import jax.numpy as jnp
from jax import lax
from jax.experimental import pallas as pl
from jax.experimental.pallas import tpu as pltpu
from jax.experimental.pallas import tpu_sc as plsc

N_NODES = 10000
N_EDGES = 160000
D_FEAT = 256
D_EDGE = 16
D_GLOBAL = 128

NC = 2
NS = 16
NW = NC * NS
CHUNK = 128
N_CHUNKS = N_EDGES // CHUNK
CHUNKS_FULL = 40
CHUNKS_LAST = N_CHUNKS - (NW - 1) * CHUNKS_FULL
SLAB = CHUNKS_FULL * CHUNK
N_PAD = 10240
ROWS_PER_TILE = N_PAD // NS


def _sc_segment_sum(edgesT, receivers, zeros):
    mesh = plsc.VectorSubcoreMesh(core_axis_name="c", subcore_axis_name="s")

    @functools.partial(
        pl.kernel,
        out_type=jax.ShapeDtypeStruct((NC, N_PAD, D_EDGE), jnp.float32),
        mesh=mesh,
        scratch_types=[
            pltpu.VMEM((D_EDGE, SLAB), jnp.float32),
            pltpu.VMEM((2, CHUNK, D_EDGE), jnp.float32),
            pltpu.VMEM((SLAB,), jnp.int32),
            pltpu.VMEM_SHARED((N_PAD, D_EDGE), jnp.float32),
            pltpu.SemaphoreType.DMA,
            pltpu.SemaphoreType.DMA,
            pltpu.SemaphoreType.DMA,
        ],
        compiler_params=pltpu.CompilerParams(
            use_tc_tiling_on_sc=False, needs_layout_passes=False
        ),
    )
    def k(edgesT_hbm, recv_hbm, zeros_hbm, out_hbm, et_v, tbuf, idx_v,
          acc_sh, stage_sem, ssem0, ssem1):
        c = lax.axis_index("c")
        s = lax.axis_index("s")
        wid = s * NC + c
        is_last = wid == NW - 1
        n_chunks = jnp.where(is_last, CHUNKS_LAST, CHUNKS_FULL)
        lanes = lax.iota(jnp.int32, D_EDGE)

        zero_cp = pltpu.async_copy(
            zeros_hbm.at[pl.ds(s * ROWS_PER_TILE, ROWS_PER_TILE)],
            acc_sh.at[pl.ds(s * ROWS_PER_TILE, ROWS_PER_TILE)],
            stage_sem,
        )

        @pl.when(jnp.logical_not(is_last))
        def _():
            e_cp = pltpu.async_copy(
                edgesT_hbm.at[:, pl.ds(wid * SLAB, SLAB)], et_v, stage_sem
            )
            i_cp = pltpu.async_copy(
                recv_hbm.at[pl.ds(wid * SLAB, SLAB)], idx_v, stage_sem
            )
            e_cp.wait()
            i_cp.wait()

        @pl.when(is_last)
        def _():
            e_cp = pltpu.async_copy(
                edgesT_hbm.at[:, pl.ds(wid * SLAB, CHUNKS_LAST * CHUNK)],
                et_v.at[:, pl.ds(0, CHUNKS_LAST * CHUNK)],
                stage_sem,
            )
            i_cp = pltpu.async_copy(
                recv_hbm.at[pl.ds(wid * SLAB, CHUNKS_LAST * CHUNK)],
                idx_v.at[pl.ds(0, CHUNKS_LAST * CHUNK)],
                stage_sem,
            )
            e_cp.wait()
            i_cp.wait()

        zero_cp.wait()

        plsc.subcore_barrier()

        def scat_wait(parity):
            sem = ssem0 if parity == 0 else ssem1
            pltpu.make_async_copy(
                tbuf.at[parity],
                acc_sh.at[idx_v.at[pl.ds(0, CHUNK)]],
                sem,
            ).wait()

        @pl.loop(0, n_chunks)
        def _(j):
            p = j & 1

            @pl.when(j >= 2)
            def _():
                @pl.when(p == 0)
                def _():
                    scat_wait(0)

                @pl.when(p == 1)
                def _():
                    scat_wait(1)

            @pl.loop(0, CHUNK)
            def _(e):
                col = jnp.full((D_EDGE,), j * CHUNK + e, jnp.int32)
                tbuf[p, e, :] = plsc.load_gather(et_v, [lanes, col])

            @pl.when(p == 0)
            def _():
                pltpu.async_copy(
                    tbuf.at[0],
                    acc_sh.at[idx_v.at[pl.ds(j * CHUNK, CHUNK)]],
                    ssem0, add=True,
                )

            @pl.when(p == 1)
            def _():
                pltpu.async_copy(
                    tbuf.at[1],
                    acc_sh.at[idx_v.at[pl.ds(j * CHUNK, CHUNK)]],
                    ssem1, add=True,
                )

        scat_wait(0)
        scat_wait(1)

        plsc.subcore_barrier()

        pltpu.sync_copy(
            acc_sh.at[pl.ds(s * ROWS_PER_TILE, ROWS_PER_TILE)],
            out_hbm.at[c, pl.ds(s * ROWS_PER_TILE, ROWS_PER_TILE)],
        )

    return k(edgesT, receivers, zeros)


BLK = 1000


def _tc_mlp(partials, nodes, globals_, W, b2):
    def body(p_ref, nodes_ref, g_ref, w_ref, b_ref, o_ref):
        agg = p_ref[0] + p_ref[1]
        w = w_ref[...]
        acc = jnp.dot(agg, w[:D_EDGE], preferred_element_type=jnp.float32)
        acc += jnp.dot(nodes_ref[...], w[D_EDGE:D_EDGE + D_FEAT],
                       preferred_element_type=jnp.float32)
        acc += jnp.dot(g_ref[...], w[D_EDGE + D_FEAT:],
                       preferred_element_type=jnp.float32)
        o_ref[...] = jnp.maximum(acc + b_ref[...], 0.0)

    return pl.pallas_call(
        body,
        grid=(N_NODES // BLK,),
        in_specs=[
            pl.BlockSpec((NC, BLK, D_EDGE), lambda i: (0, i, 0)),
            pl.BlockSpec((BLK, D_FEAT), lambda i: (i, 0)),
            pl.BlockSpec((1, D_GLOBAL), lambda i: (0, 0)),
            pl.BlockSpec((D_EDGE + D_FEAT + D_GLOBAL, D_FEAT),
                         lambda i: (0, 0)),
            pl.BlockSpec((1, D_FEAT), lambda i: (0, 0)),
        ],
        out_specs=pl.BlockSpec((BLK, D_FEAT), lambda i: (i, 0)),
        out_shape=jax.ShapeDtypeStruct((N_NODES, D_FEAT), jnp.float32),
    )(partials, nodes, globals_, W, b2)


def kernel(nodes, edges, receivers, senders, globals_, W, b):
    del senders
    zeros = jnp.zeros((N_PAD, D_EDGE), jnp.float32)
    partials = _sc_segment_sum(edges.T, receivers, zeros)
    return _tc_mlp(partials, nodes, globals_, W, b.reshape(1, D_FEAT))
